# Initial kernel scaffold; baseline (speedup 1.0000x reference)
import jax
import jax.numpy as jnp
from jax.experimental import pallas as pl


def kernel(table, indices):
    raise NotImplementedError("write your pallas kernel here")


# speedup vs baseline: 9.0813x; 9.0813x over previous
import functools

import jax
import jax.numpy as jnp
from jax import lax
from jax.experimental import pallas as pl
from jax.experimental.pallas import tpu as pltpu
from jax.experimental.pallas import tpu_sc as plsc

VOCAB = 400000
D = 50
DP = 64
B = 4096
L = 350
LP = 352
NW = 32
RPT = B // NW
INV_L = 1.0 / L


def _glove_sc_body(table_hbm, idx_hbm, out_hbm, idx_v, rows_v, stage_v, sem):
    wid = lax.axis_index("s") * 2 + lax.axis_index("c")
    base = wid * RPT

    pltpu.sync_copy(idx_hbm.at[pl.ds(base * LP, RPT * LP)], idx_v)

    def row_body(b, carry):
        off = b * LP
        cp0 = pltpu.async_copy(
            table_hbm.at[idx_v.at[pl.ds(off, 128)]],
            rows_v.at[pl.ds(0, 128)], sem)
        cp1 = pltpu.async_copy(
            table_hbm.at[idx_v.at[pl.ds(off + 128, 128)]],
            rows_v.at[pl.ds(128, 128)], sem)
        cp2 = pltpu.async_copy(
            table_hbm.at[idx_v.at[pl.ds(off + 256, 96)]],
            rows_v.at[pl.ds(256, 96)], sem)
        cp0.wait()
        cp1.wait()
        cp2.wait()

        def tok(t, acc):
            a0, a1, a2, a3 = acc
            return (a0 + rows_v[t, pl.ds(0, 16)],
                    a1 + rows_v[t, pl.ds(16, 16)],
                    a2 + rows_v[t, pl.ds(32, 16)],
                    a3 + rows_v[t, pl.ds(34, 16)])

        z = jnp.zeros((16,), jnp.float32)
        a0, a1, a2, a3 = lax.fori_loop(0, L, tok, (z, z, z, z))
        s = jnp.float32(INV_L)
        stage_v[b, pl.ds(0, 16)] = a0 * s
        stage_v[b, pl.ds(16, 16)] = a1 * s
        stage_v[b, pl.ds(32, 16)] = a2 * s
        stage_v[b, pl.ds(34, 16)] = a3 * s
        return carry

    lax.fori_loop(0, RPT, row_body, 0)
    pltpu.sync_copy(stage_v, out_hbm.at[pl.ds(base, RPT)])


_glove_sc = pl.kernel(
    _glove_sc_body,
    out_type=jax.ShapeDtypeStruct((B, D), jnp.float32),
    mesh=plsc.VectorSubcoreMesh(core_axis_name="c", subcore_axis_name="s"),
    compiler_params=pltpu.CompilerParams(use_tc_tiling_on_sc=False),
    scratch_types=[
        pltpu.VMEM((RPT * LP,), jnp.int32),
        pltpu.VMEM((LP, DP), jnp.float32),
        pltpu.VMEM((RPT, D), jnp.float32),
        pltpu.SemaphoreType.DMA,
    ],
)


def kernel(table, indices):
    table_p = jnp.pad(table, ((0, 0), (0, DP - D)))
    idx32 = indices.astype(jnp.int32)
    idx_flat = jnp.pad(idx32, ((0, 0), (0, LP - L))).reshape(-1)
    return _glove_sc(table_p, idx_flat)


# speedup vs baseline: 9.8107x; 1.0803x over previous
import jax
import jax.numpy as jnp
from jax import lax
from jax.experimental import pallas as pl
from jax.experimental.pallas import tpu as pltpu
from jax.experimental.pallas import tpu_sc as plsc

VOCAB = 400000
D = 50
DP = 64
B = 4096
L = 350
LP = 352
NW = 32
RPT = B // NW
INV_L = 1.0 / L
UNROLL = 5


def _glove_sc_body(table_hbm, idx_hbm, out_hbm, idx_v, rows0_v, rows1_v,
                   stage_v, sem0, sem1):
    wid = lax.axis_index("s") * 2 + lax.axis_index("c")
    base = wid * RPT

    pltpu.sync_copy(idx_hbm.at[pl.ds(base * LP, RPT * LP)], idx_v)

    def issue(b, buf, sem):
        off = b * LP
        pltpu.async_copy(table_hbm.at[idx_v.at[pl.ds(off, 128)]],
                         buf.at[pl.ds(0, 128)], sem)
        pltpu.async_copy(table_hbm.at[idx_v.at[pl.ds(off + 128, 128)]],
                         buf.at[pl.ds(128, 128)], sem)
        pltpu.async_copy(table_hbm.at[idx_v.at[pl.ds(off + 256, 96)]],
                         buf.at[pl.ds(256, 96)], sem)

    def drain(buf, sem):
        pltpu.make_async_copy(table_hbm.at[pl.ds(0, LP)], buf, sem).wait()

    def reduce_into(buf, b):
        def tok(t, acc):
            a0, a1, a2, a3, b0, b1, b2, b3 = acc
            t0 = t * UNROLL
            a0 += buf[t0, pl.ds(0, 16)]
            a1 += buf[t0, pl.ds(16, 16)]
            a2 += buf[t0, pl.ds(32, 16)]
            a3 += buf[t0, pl.ds(34, 16)]
            b0 += buf[t0 + 1, pl.ds(0, 16)]
            b1 += buf[t0 + 1, pl.ds(16, 16)]
            b2 += buf[t0 + 1, pl.ds(32, 16)]
            b3 += buf[t0 + 1, pl.ds(34, 16)]
            a0 += buf[t0 + 2, pl.ds(0, 16)]
            a1 += buf[t0 + 2, pl.ds(16, 16)]
            a2 += buf[t0 + 2, pl.ds(32, 16)]
            a3 += buf[t0 + 2, pl.ds(34, 16)]
            b0 += buf[t0 + 3, pl.ds(0, 16)]
            b1 += buf[t0 + 3, pl.ds(16, 16)]
            b2 += buf[t0 + 3, pl.ds(32, 16)]
            b3 += buf[t0 + 3, pl.ds(34, 16)]
            a0 += buf[t0 + 4, pl.ds(0, 16)]
            a1 += buf[t0 + 4, pl.ds(16, 16)]
            a2 += buf[t0 + 4, pl.ds(32, 16)]
            a3 += buf[t0 + 4, pl.ds(34, 16)]
            return (a0, a1, a2, a3, b0, b1, b2, b3)

        z = jnp.zeros((16,), jnp.float32)
        a0, a1, a2, a3, b0, b1, b2, b3 = lax.fori_loop(
            0, L // UNROLL, tok, (z,) * 8)
        s = jnp.float32(INV_L)
        stage_v[b, pl.ds(0, 16)] = (a0 + b0) * s
        stage_v[b, pl.ds(16, 16)] = (a1 + b1) * s
        stage_v[b, pl.ds(32, 16)] = (a2 + b2) * s
        stage_v[b, pl.ds(34, 16)] = (a3 + b3) * s

    issue(0, rows0_v, sem0)

    def pair_body(i, carry):
        b0 = 2 * i
        issue(b0 + 1, rows1_v, sem1)
        drain(rows0_v, sem0)
        reduce_into(rows0_v, b0)

        @pl.when(b0 + 2 < RPT)
        def _():
            issue(b0 + 2, rows0_v, sem0)

        drain(rows1_v, sem1)
        reduce_into(rows1_v, b0 + 1)
        return carry

    lax.fori_loop(0, RPT // 2, pair_body, 0)
    pltpu.sync_copy(stage_v, out_hbm.at[pl.ds(base, RPT)])


_glove_sc = pl.kernel(
    _glove_sc_body,
    out_type=jax.ShapeDtypeStruct((B, D), jnp.float32),
    mesh=plsc.VectorSubcoreMesh(core_axis_name="c", subcore_axis_name="s"),
    compiler_params=pltpu.CompilerParams(use_tc_tiling_on_sc=False),
    scratch_types=[
        pltpu.VMEM((RPT * LP,), jnp.int32),
        pltpu.VMEM((LP, DP), jnp.float32),
        pltpu.VMEM((LP, DP), jnp.float32),
        pltpu.VMEM((RPT, D), jnp.float32),
        pltpu.SemaphoreType.DMA,
        pltpu.SemaphoreType.DMA,
    ],
)


def kernel(table, indices):
    table_p = jnp.pad(table, ((0, 0), (0, DP - D)))
    idx32 = indices.astype(jnp.int32)
    idx_flat = jnp.pad(idx32, ((0, 0), (0, LP - L))).reshape(-1)
    return _glove_sc(table_p, idx_flat)
